# Initial kernel scaffold; baseline (speedup 1.0000x reference)
import jax
import jax.numpy as jnp
from jax.experimental import pallas as pl


def kernel(tokens, table):
    raise NotImplementedError("write your pallas kernel here")


# speedup vs baseline: 2.2462x; 2.2462x over previous
import functools
import jax
import jax.numpy as jnp
from jax import lax
from jax.experimental import pallas as pl
from jax.experimental.pallas import tpu as pltpu
from jax.experimental.pallas import tpu_sc as plsc

BATCH = 1024
SEQ = 512
D = 512
NTOK = BATCH * SEQ
NC, NS = 2, 16
NW = NC * NS
PER_W = NTOK // NW
CHUNK = 64
NCHUNK = PER_W // CHUNK


def _embed_body(tok_hbm, table_hbm, out_hbm, idx_v, buf0, buf1, g0, g1, o0, o1):
    wid = lax.axis_index("s") * NC + lax.axis_index("c")
    base = wid * PER_W
    pltpu.sync_copy(tok_hbm.at[wid], idx_v)

    bufs = (buf0, buf1)
    gsems = (g0, g1)
    osems = (o0, o1)

    def gather(c, b):
        return pltpu.make_async_copy(table_hbm.at[idx_v.at[c]], bufs[b], gsems[b])

    def out_copy(c, b):
        return pltpu.make_async_copy(
            bufs[b], out_hbm.at[pl.ds(base + c * CHUNK, CHUNK)], osems[b])

    gather(0, 0).start()
    gather(0, 0).wait()
    out_copy(0, 0).start()
    gather(1, 1).start()
    gather(1, 1).wait()
    out_copy(1, 1).start()
    out_copy(0, 0).wait()
    gather(2, 0).start()

    @pl.loop(1, NCHUNK // 2 - 1)
    def _(p):
        c = 2 * p
        gather(c, 0).wait()
        out_copy(c, 0).start()
        out_copy(c - 1, 1).wait()
        gather(c + 1, 1).start()
        gather(c + 1, 1).wait()
        out_copy(c + 1, 1).start()
        out_copy(c, 0).wait()
        gather(c + 2, 0).start()

    c = NCHUNK - 2
    gather(c, 0).wait()
    out_copy(c, 0).start()
    out_copy(c - 1, 1).wait()
    gather(c + 1, 1).start()
    gather(c + 1, 1).wait()
    out_copy(c + 1, 1).start()
    out_copy(c, 0).wait()
    out_copy(c + 1, 1).wait()


@jax.jit
def _embed(tok, table):
    run = functools.partial(
        pl.kernel,
        out_type=jax.ShapeDtypeStruct((NTOK, D), jnp.float32),
        mesh=plsc.VectorSubcoreMesh(
            core_axis_name="c", subcore_axis_name="s",
            num_cores=NC, num_subcores=NS),
        scratch_types=[
            pltpu.VMEM((NCHUNK, CHUNK), jnp.int32),
            pltpu.VMEM((CHUNK, D), jnp.float32),
            pltpu.VMEM((CHUNK, D), jnp.float32),
            pltpu.SemaphoreType.DMA,
            pltpu.SemaphoreType.DMA,
            pltpu.SemaphoreType.DMA,
            pltpu.SemaphoreType.DMA,
        ],
    )(_embed_body)
    return run(tok, table)


def kernel(tokens, table):
    tok = tokens.astype(jnp.int32).reshape(NW, NCHUNK, CHUNK)
    out = _embed(tok, table)
    return out.reshape(BATCH, SEQ, D)


# speedup vs baseline: 2.2833x; 1.0165x over previous
import functools
import jax
import jax.numpy as jnp
from jax import lax
from jax.experimental import pallas as pl
from jax.experimental.pallas import tpu as pltpu
from jax.experimental.pallas import tpu_sc as plsc

BATCH = 1024
SEQ = 512
D = 512
NTOK = BATCH * SEQ
NC, NS = 2, 16
NW = NC * NS
PER_W = NTOK // NW
CHUNK = 64
NCHUNK = PER_W // CHUNK
NBUF = 3


def _embed_body(tok_hbm, table_hbm, out_hbm, idx_v,
                buf0, buf1, buf2, g0, g1, g2, o0, o1, o2):
    wid = lax.axis_index("s") * NC + lax.axis_index("c")
    base = wid * PER_W
    pltpu.sync_copy(tok_hbm.at[wid], idx_v)

    bufs = (buf0, buf1, buf2)
    gsems = (g0, g1, g2)
    osems = (o0, o1, o2)

    def gather(c, b):
        return pltpu.make_async_copy(table_hbm.at[idx_v.at[c]], bufs[b], gsems[b])

    def out_copy(c, b):
        return pltpu.make_async_copy(
            bufs[b], out_hbm.at[pl.ds(base + c * CHUNK, CHUNK)], osems[b])

    def step(c, b):
        gather(c, b).wait()
        out_copy(c, b).start()
        out_copy(c - 1, (b + 2) % NBUF).wait()
        gather(c + 2, (b + 2) % NBUF).start()

    gather(0, 0).start()
    gather(1, 1).start()
    gather(0, 0).wait()
    out_copy(0, 0).start()
    gather(2, 2).start()
    step(1, 1)

    @pl.loop(2, NCHUNK - 2, step=NBUF)
    def _(c):
        step(c, 2)
        step(c + 1, 0)
        step(c + 2, 1)

    c = NCHUNK - 2
    gather(c, 2).wait()
    out_copy(c, 2).start()
    out_copy(c - 1, 1).wait()
    gather(c + 1, 0).wait()
    out_copy(c + 1, 0).start()
    out_copy(c, 2).wait()
    out_copy(c + 1, 0).wait()


@jax.jit
def _embed(tok, table):
    run = functools.partial(
        pl.kernel,
        out_type=jax.ShapeDtypeStruct((NTOK, D), jnp.float32),
        mesh=plsc.VectorSubcoreMesh(
            core_axis_name="c", subcore_axis_name="s",
            num_cores=NC, num_subcores=NS),
        scratch_types=[
            pltpu.VMEM((NCHUNK, CHUNK), jnp.int32),
            pltpu.VMEM((CHUNK, D), jnp.float32),
            pltpu.VMEM((CHUNK, D), jnp.float32),
            pltpu.VMEM((CHUNK, D), jnp.float32),
            pltpu.SemaphoreType.DMA,
            pltpu.SemaphoreType.DMA,
            pltpu.SemaphoreType.DMA,
            pltpu.SemaphoreType.DMA,
            pltpu.SemaphoreType.DMA,
            pltpu.SemaphoreType.DMA,
        ],
    )(_embed_body)
    return run(tok, table)


def kernel(tokens, table):
    tok = tokens.astype(jnp.int32).reshape(NW, NCHUNK, CHUNK)
    out = _embed(tok, table)
    return out.reshape(BATCH, SEQ, D)


# speedup vs baseline: 2.2925x; 1.0040x over previous
import functools
import jax
import jax.numpy as jnp
from jax import lax
from jax.experimental import pallas as pl
from jax.experimental.pallas import tpu as pltpu
from jax.experimental.pallas import tpu_sc as plsc

BATCH = 1024
SEQ = 512
D = 512
NTOK = BATCH * SEQ
NC, NS = 2, 16
NW = NC * NS
PER_W = NTOK // NW
CHUNK = 96
NFULL = 170
TAIL = PER_W - NFULL * CHUNK


def _embed_body(tok_hbm, table_hbm, out_hbm, idx_v, buf0, buf1, g0, g1, o0, o1):
    wid = lax.axis_index("s") * NC + lax.axis_index("c")
    base = wid * PER_W
    pltpu.sync_copy(tok_hbm.at[wid], idx_v)

    bufs = (buf0, buf1)
    gsems = (g0, g1)
    osems = (o0, o1)

    def gather(c, b):
        return pltpu.make_async_copy(
            table_hbm.at[idx_v.at[pl.ds(c * CHUNK, CHUNK)]], bufs[b], gsems[b])

    def out_copy(c, b):
        return pltpu.make_async_copy(
            bufs[b], out_hbm.at[pl.ds(base + c * CHUNK, CHUNK)], osems[b])

    gather(0, 0).start()
    gather(0, 0).wait()
    out_copy(0, 0).start()
    gather(1, 1).start()
    gather(1, 1).wait()
    out_copy(1, 1).start()
    out_copy(0, 0).wait()
    gather(2, 0).start()

    @pl.loop(1, NFULL // 2 - 1)
    def _(p):
        c = 2 * p
        gather(c, 0).wait()
        out_copy(c, 0).start()
        out_copy(c - 1, 1).wait()
        gather(c + 1, 1).start()
        gather(c + 1, 1).wait()
        out_copy(c + 1, 1).start()
        out_copy(c, 0).wait()
        gather(c + 2, 0).start()

    c = NFULL - 2
    gather(c, 0).wait()
    out_copy(c, 0).start()
    out_copy(c - 1, 1).wait()
    gather(c + 1, 1).start()
    gather(c + 1, 1).wait()
    out_copy(c + 1, 1).start()
    out_copy(c, 0).wait()
    tail_g = pltpu.make_async_copy(
        table_hbm.at[idx_v.at[pl.ds(NFULL * CHUNK, TAIL)]],
        buf0.at[pl.ds(0, TAIL)], gsems[0])
    tail_o = pltpu.make_async_copy(
        buf0.at[pl.ds(0, TAIL)],
        out_hbm.at[pl.ds(base + NFULL * CHUNK, TAIL)], osems[0])
    tail_g.start()
    tail_g.wait()
    tail_o.start()
    out_copy(c + 1, 1).wait()
    tail_o.wait()


@jax.jit
def _embed(tok, table):
    run = functools.partial(
        pl.kernel,
        out_type=jax.ShapeDtypeStruct((NTOK, D), jnp.float32),
        mesh=plsc.VectorSubcoreMesh(
            core_axis_name="c", subcore_axis_name="s",
            num_cores=NC, num_subcores=NS),
        scratch_types=[
            pltpu.VMEM((PER_W,), jnp.int32),
            pltpu.VMEM((CHUNK, D), jnp.float32),
            pltpu.VMEM((CHUNK, D), jnp.float32),
            pltpu.SemaphoreType.DMA,
            pltpu.SemaphoreType.DMA,
            pltpu.SemaphoreType.DMA,
            pltpu.SemaphoreType.DMA,
        ],
    )(_embed_body)
    return run(tok, table)


def kernel(tokens, table):
    tok = tokens.astype(jnp.int32).reshape(NW, PER_W)
    out = _embed(tok, table)
    return out.reshape(BATCH, SEQ, D)
